# baseline (speedup 1.0000x reference)
import functools

import jax
import jax.numpy as jnp
from jax import lax
from jax.experimental import pallas as pl
from jax.experimental.pallas import tpu as pltpu
from jax.experimental.pallas import tpu_sc as plsc

NB = (64, 32)
NG = 5000
NR = 16
N_CUTS = 500000

_NC = 2
_NS = 16
NW = _NC * _NS
CUTS_PAD = 512000
PER_TILE = CUTS_PAD // NW
CHUNK = 3200
NCHUNK = PER_TILE // CHUNK
GSZ = 128
NDMA = CHUNK // GSZ
NGRP = CHUNK // 16

BG1 = 200
BG3 = 200


def _csum_last(a):
    n = a.shape[-1]
    k = 1
    while k < n:
        sh = jnp.concatenate(
            [jnp.zeros_like(a[..., :k]), a[..., :-k]], axis=-1)
        a = a + sh
        k *= 2
    return a


def _softmax_last(u):
    m = jnp.max(u, axis=-1, keepdims=True)
    e = jnp.exp(u - m)
    return e / jnp.sum(e, axis=-1, keepdims=True)


def _k1_body(uw_ref, uh_ref, dh_ref, cumw0_ref, cumw1_ref, a0_ref, a1_ref):
    uw = uw_ref[...]
    uh = uh_ref[...]
    dh = dh_ref[...]
    woff = 0
    hoff = 0
    for li, n in enumerate(NB):
        uwl = uw[:, woff:woff + n - 1]
        sw = _softmax_last(uwl)
        incl = _csum_last(sw)
        cumw = jnp.concatenate(
            [jnp.zeros_like(sw[:, :1]), incl], axis=-1)
        e = jnp.exp(uh[None, :, hoff:hoff + n] + dh[:, :, hoff:hoff + n])
        pa = (e[..., :-1] + e[..., 1:]) * 0.5 * sw[None]
        A = jnp.sum(pa, axis=-1)
        if li == 0:
            cumw0_ref[...] = cumw
            a0_ref[...] = A[..., None]
        else:
            cumw1_ref[...] = cumw
            a1_ref[...] = A[..., None]
        woff += n - 1
        hoff += n


def _k1_call(uw, uh, dh):
    grid = (NG // BG1,)
    return pl.pallas_call(
        _k1_body,
        grid=grid,
        in_specs=[
            pl.BlockSpec((BG1, 94), lambda gb: (gb, 0)),
            pl.BlockSpec((BG1, 96), lambda gb: (gb, 0)),
            pl.BlockSpec((NR, BG1, 96), lambda gb: (0, gb, 0)),
        ],
        out_specs=[
            pl.BlockSpec((BG1, NB[0]), lambda gb: (gb, 0)),
            pl.BlockSpec((BG1, NB[1]), lambda gb: (gb, 0)),
            pl.BlockSpec((NR, BG1, 1), lambda gb: (0, gb, 0)),
            pl.BlockSpec((NR, BG1, 1), lambda gb: (0, gb, 0)),
        ],
        out_shape=[
            jax.ShapeDtypeStruct((NG, NB[0]), jnp.float32),
            jax.ShapeDtypeStruct((NG, NB[1]), jnp.float32),
            jax.ShapeDtypeStruct((NR, NG, 1), jnp.float32),
            jax.ShapeDtypeStruct((NR, NG, 1), jnp.float32),
        ],
    )(uw, uh, dh)


def _k2_body(a0_ref, a1_ref, cp0_ref, m0_ref, cp1_ref, s1_ref,
             ia0_ref, ia1_ref):
    A0 = a0_ref[...]
    GA0 = A0 * jnp.float32(1.0 / NG)
    area0 = jnp.sum(GA0, axis=-1, keepdims=True)
    M0 = GA0 / area0
    incl0 = _csum_last(M0)
    CP0 = jnp.concatenate(
        [jnp.zeros_like(M0[:, :1]), incl0[:, :-1]], axis=-1)
    A1 = a1_ref[...]
    G1 = M0
    GA1 = G1 * A1
    area1 = jnp.sum(GA1, axis=-1, keepdims=True)
    M1 = GA1 / area1
    incl1 = _csum_last(M1)
    CP1 = jnp.concatenate(
        [jnp.zeros_like(M1[:, :1]), incl1[:, :-1]], axis=-1)
    S1 = G1 / area1
    cp0_ref[...] = CP0
    m0_ref[...] = M0
    cp1_ref[...] = CP1
    s1_ref[...] = S1
    ia0_ref[...] = 1.0 / area0
    ia1_ref[...] = 1.0 / area1


def _k2_call(A0, A1):
    return pl.pallas_call(
        _k2_body,
        out_shape=[
            jax.ShapeDtypeStruct((NR, NG), jnp.float32),
            jax.ShapeDtypeStruct((NR, NG), jnp.float32),
            jax.ShapeDtypeStruct((NR, NG), jnp.float32),
            jax.ShapeDtypeStruct((NR, NG), jnp.float32),
            jax.ShapeDtypeStruct((NR, 1), jnp.float32),
            jax.ShapeDtypeStruct((NR, 1), jnp.float32),
        ],
    )(A0, A1)


def _k3_body(uw_ref, uh_ref, dh_ref, cumw0_ref, cumw1_ref,
             cp0_ref, s0_ref, cp1_ref, s1_ref, h0s_ref, h1s_ref,
             q0_ref, q1_ref):
    uw = uw_ref[...]
    uh = uh_ref[...]
    dh = dh_ref[...]
    woff = 0
    hoff = 0
    for li, n in enumerate(NB):
        uwl = uw[:, woff:woff + n - 1]
        sw = _softmax_last(uwl)
        e = jnp.exp(uh[:, hoff:hoff + n] + dh[:, hoff:hoff + n])
        pa = (e[:, :-1] + e[:, 1:]) * 0.5 * sw
        C = jnp.concatenate(
            [jnp.zeros_like(e[:, :1]), _csum_last(pa)], axis=-1)
        if li == 0:
            cumw = cumw0_ref[...]
            cp = cp0_ref[...]
            s = s0_ref[...]
            hs = h0s_ref[...]
        else:
            cumw = cumw1_ref[...]
            cp = cp1_ref[...]
            s = s1_ref[...]
            hs = h1s_ref[...]
        h = e * hs
        lc = cp + s * C
        cwn = jnp.concatenate([cumw[:, 1:], cumw[:, -1:] + 1.0], axis=-1)
        hn = jnp.concatenate([h[:, 1:], h[:, -1:]], axis=-1)
        z = jnp.zeros_like(h)
        q = jnp.stack([cumw, cwn, h, hn, lc, z, z, z], axis=-1)
        if li == 0:
            q0_ref[...] = q
        else:
            q1_ref[...] = q
        woff += n - 1
        hoff += n


def _k3_call(uw, uh, dh, cumw0, cumw1, cp0x, s0x, cp1x, s1x, h0sx, h1sx):
    grid = (NR, NG // BG3)
    pgx = lambda r, gb: (gb, 0)
    prgx = lambda r, gb: (r, gb, 0)
    return pl.pallas_call(
        _k3_body,
        grid=grid,
        in_specs=[
            pl.BlockSpec((BG3, 94), pgx),
            pl.BlockSpec((BG3, 96), pgx),
            pl.BlockSpec((None, BG3, 96), prgx),
            pl.BlockSpec((BG3, NB[0]), pgx),
            pl.BlockSpec((BG3, NB[1]), pgx),
            pl.BlockSpec((None, BG3, 1), prgx),
            pl.BlockSpec((None, BG3, 1), prgx),
            pl.BlockSpec((None, BG3, 1), prgx),
            pl.BlockSpec((None, BG3, 1), prgx),
            pl.BlockSpec((None, BG3, 1), prgx),
            pl.BlockSpec((None, BG3, 1), prgx),
        ],
        out_specs=[
            pl.BlockSpec((None, BG3, NB[0], 8), lambda r, gb: (r, gb, 0, 0)),
            pl.BlockSpec((None, BG3, NB[1], 8), lambda r, gb: (r, gb, 0, 0)),
        ],
        out_shape=[
            jax.ShapeDtypeStruct((NR, NG, NB[0], 8), jnp.float32),
            jax.ShapeDtypeStruct((NR, NG, NB[1], 8), jnp.float32),
        ],
    )(uw, uh, dh, cumw0, cumw1, cp0x, s0x, cp1x, s1x, h0sx, h1sx)


def _sc_body(x_hbm, rxg_hbm, q0_hbm, q1_hbm, pg_hbm, out_hbm, dp_hbm,
             xv, rxgv, idxv, rows, pgv, outv, dpv, rows16, sem, sem2):
    c = lax.axis_index("c")
    s = lax.axis_index("s")
    wid = s * _NC + c
    tbase = wid * PER_TILE
    i16 = lax.iota(jnp.int32, 16)

    def fields_from(rref, lanes):
        return [
            plsc.load_gather(rref, [lanes, jnp.full((16,), ci, jnp.int32)])
            for ci in range(5)
        ]

    def level(lv, n, q_hbm):
        nm1 = jnp.float32(n - 1)
        jmaxi = n - 2

        def pg_of(gi, rxg):
            if lv == 0:
                gq = rxg % NG
                P = gq.astype(jnp.float32) * jnp.float32(1.0 / NG)
                G = jnp.full((16,), 1.0 / NG, jnp.float32)
            else:
                lanes = gi * 16 + i16
                P = plsc.load_gather(pgv, [lanes, jnp.zeros((16,), jnp.int32)])
                G = plsc.load_gather(pgv, [lanes, jnp.ones((16,), jnp.int32)])
            return P, G

        def guess(t):
            tj = jnp.clip(t * nm1, 0.0, jnp.float32(jmaxi))
            return tj.astype(jnp.int32)

        def passA(gi, carry):
            sl = pl.ds(gi * 16, 16)
            rxg = rxgv[sl]
            xx = xv[sl] if lv == 0 else outv[sl]
            P, G = pg_of(gi, rxg)
            t = (xx - P) / G
            j = guess(t)
            idxv[gi // 8, pl.ds((gi % 8) * 16, 16)] = rxg * n + j
            return carry

        lax.fori_loop(0, NGRP, passA, 0)

        cps = [
            pltpu.async_copy(q_hbm.at[idxv.at[si]],
                             rows.at[pl.ds(si * GSZ, GSZ)], sem)
            for si in range(NDMA)
        ]
        for cp_ in cps:
            cp_.wait()

        def passB(gi, carry):
            sl = pl.ds(gi * 16, 16)
            lanes = gi * 16 + i16
            rxg = rxgv[sl]
            xx = xv[sl] if lv == 0 else outv[sl]
            P, G = pg_of(gi, rxg)
            t = (xx - P) / G
            j = guess(t)
            cw, cw1, hj, hj1, lcv = fields_from(rows, lanes)

            def wr(j_, cw_, cw1_):
                return (((cw_ >= t) & (j_ > 0))
                        | ((cw1_ < t) & (j_ < jmaxi)))

            wrong = wr(j, cw, cw1)

            def cond(st):
                return jnp.any(st[6])

            def body(st):
                j_, cw_, cw1_, hj_, hj1_, lc_, w_ = st
                jn = jnp.where(w_, jnp.where(cw_ >= t, j_ - 1, j_ + 1), j_)
                pltpu.async_copy(q_hbm.at[rxg * n + jn], rows16, sem2).wait()
                nf = fields_from(rows16, i16)
                cwN = jnp.where(w_, nf[0], cw_)
                cw1N = jnp.where(w_, nf[1], cw1_)
                hjN = jnp.where(w_, nf[2], hj_)
                hj1N = jnp.where(w_, nf[3], hj1_)
                lcN = jnp.where(w_, nf[4], lc_)
                return (jn, cwN, cw1N, hjN, hj1N, lcN, wr(jn, cwN, cw1N))

            j, cw, cw1, hj, hj1, lcv, wrong = lax.while_loop(
                cond, body, (j, cw, cw1, hj, hj1, lcv, wrong))
            ll = P + G * cw
            bw = G * (cw1 - cw)
            alpha = jnp.clip(
                (xx - ll) / jnp.maximum(bw, jnp.float32(1e-12)), 0.0, 1.0)
            outn = (lcv + alpha * bw * hj
                    + jnp.float32(0.5) * alpha * alpha * bw * (hj1 - hj))
            dd = jnp.maximum(hj + alpha * (hj1 - hj), jnp.float32(1e-12))
            outv[sl] = outn
            if lv == 0:
                dpv[sl] = dd
            else:
                dpv[sl] = dpv[sl] * dd
            return carry

        lax.fori_loop(0, NGRP, passB, 0)

    def chunk(ci, carry):
        base = tbase + ci * CHUNK
        pltpu.sync_copy(x_hbm.at[pl.ds(base, CHUNK)], xv)
        pltpu.sync_copy(rxg_hbm.at[pl.ds(base, CHUNK)], rxgv)
        level(0, NB[0], q0_hbm)

        def pgidx(gi, carry):
            idxv[gi // 8, pl.ds((gi % 8) * 16, 16)] = rxgv[pl.ds(gi * 16, 16)]
            return carry

        lax.fori_loop(0, NGRP, pgidx, 0)
        cps = [
            pltpu.async_copy(pg_hbm.at[idxv.at[si]],
                             pgv.at[pl.ds(si * GSZ, GSZ)], sem)
            for si in range(NDMA)
        ]
        for cp_ in cps:
            cp_.wait()
        level(1, NB[1], q1_hbm)
        pltpu.sync_copy(outv, out_hbm.at[pl.ds(base, CHUNK)])
        pltpu.sync_copy(dpv, dp_hbm.at[pl.ds(base, CHUNK)])
        return carry

    lax.fori_loop(0, NCHUNK, chunk, 0)


def _sc_call(xp, rxgp, q0, q1, pg):
    mesh = plsc.VectorSubcoreMesh(core_axis_name="c", subcore_axis_name="s")
    f = pl.kernel(
        _sc_body,
        out_type=[
            jax.ShapeDtypeStruct((CUTS_PAD,), jnp.float32),
            jax.ShapeDtypeStruct((CUTS_PAD,), jnp.float32),
        ],
        mesh=mesh,
        compiler_params=pltpu.CompilerParams(
            needs_layout_passes=False, use_tc_tiling_on_sc=False),
        scratch_types=[
            pltpu.VMEM((CHUNK,), jnp.float32),
            pltpu.VMEM((CHUNK,), jnp.int32),
            pltpu.VMEM((NDMA, GSZ), jnp.int32),
            pltpu.VMEM((CHUNK, 8), jnp.float32),
            pltpu.VMEM((CHUNK, 8), jnp.float32),
            pltpu.VMEM((CHUNK,), jnp.float32),
            pltpu.VMEM((CHUNK,), jnp.float32),
            pltpu.VMEM((16, 8), jnp.float32),
            pltpu.SemaphoreType.DMA,
            pltpu.SemaphoreType.DMA,
        ],
    )
    return f(xp, rxgp, q0, q1, pg)


def _k4_body(dp_ref, lad_ref):
    lad_ref[...] = jnp.log(dp_ref[...])


def _k4_call(dp):
    d2 = dp.reshape(CUTS_PAD // 128, 128)
    out = pl.pallas_call(
        _k4_body,
        out_shape=jax.ShapeDtypeStruct((CUTS_PAD // 128, 128), jnp.float32),
    )(d2)
    return out.reshape(CUTS_PAD)


def kernel(cut_positions, cut_local_reflatentxgene_ix, cut_local_gene_ix,
           cut_local_reflatent_ix, mixture_delta_reflatentxgene,
           unnormalized_heights, unnormalized_widths):
    uw = unnormalized_widths
    uh = unnormalized_heights
    dh = mixture_delta_reflatentxgene

    cumw0, cumw1, A0x, A1x = _k1_call(uw, uh, dh)
    A0 = A0x[..., 0]
    A1 = A1x[..., 0]
    CP0, M0, CP1, S1, ia0, ia1 = _k2_call(A0, A1)

    cp0x = CP0[..., None]
    s0x = jnp.broadcast_to(ia0 * jnp.float32(1.0 / NG), (NR, NG))[..., None]
    cp1x = CP1[..., None]
    s1x = S1[..., None]
    h0sx = jnp.broadcast_to(ia0, (NR, NG))[..., None]
    h1sx = jnp.broadcast_to(ia1, (NR, NG))[..., None]

    q0x, q1x = _k3_call(uw, uh, dh, cumw0, cumw1,
                        cp0x, s0x, cp1x, s1x, h0sx, h1sx)
    q0 = q0x.reshape(NR * NG * NB[0], 8)
    q1 = q1x.reshape(NR * NG * NB[1], 8)
    pg = jnp.pad(
        jnp.stack([CP0.reshape(-1), M0.reshape(-1)], axis=-1),
        ((0, 0), (0, 6)))

    npad = CUTS_PAD - N_CUTS
    xp = jnp.pad(cut_positions, (0, npad))
    rxgp = jnp.pad(cut_local_reflatentxgene_ix, (0, npad))

    outp, dpp = _sc_call(xp, rxgp, q0, q1, pg)
    ladp = _k4_call(dpp)
    return outp[:N_CUTS], ladp[:N_CUTS]


# speedup vs baseline: 1.2696x; 1.2696x over previous
import functools

import jax
import jax.numpy as jnp
from jax import lax
from jax.experimental import pallas as pl
from jax.experimental.pallas import tpu as pltpu
from jax.experimental.pallas import tpu_sc as plsc

NB = (64, 32)
NG = 5000
NR = 16
N_CUTS = 500000

_NC = 2
_NS = 16
NW = _NC * _NS
CUTS_PAD = 512000
PER_TILE = CUTS_PAD // NW
CHUNK = 3200
NCHUNK = PER_TILE // CHUNK
GSZ = 128
NDMA = CHUNK // GSZ
NGRP = CHUNK // 16

BG1 = 200
BG3 = 200


def _csum_last(a):
    n = a.shape[-1]
    k = 1
    while k < n:
        sh = jnp.concatenate(
            [jnp.zeros_like(a[..., :k]), a[..., :-k]], axis=-1)
        a = a + sh
        k *= 2
    return a


def _softmax_last(u):
    m = jnp.max(u, axis=-1, keepdims=True)
    e = jnp.exp(u - m)
    return e / jnp.sum(e, axis=-1, keepdims=True)


def _k1_body(uw_ref, uh_ref, dh_ref, cumw0_ref, cumw1_ref, cwn0_ref,
             cwn1_ref, a0_ref, a1_ref):
    uw = uw_ref[...]
    uh = uh_ref[...]
    dh = dh_ref[...]
    woff = 0
    hoff = 0
    for li, n in enumerate(NB):
        uwl = uw[:, woff:woff + n - 1]
        sw = _softmax_last(uwl)
        incl = _csum_last(sw)
        cumw = jnp.concatenate(
            [jnp.zeros_like(sw[:, :1]), incl], axis=-1)
        e = jnp.exp(uh[None, :, hoff:hoff + n] + dh[:, :, hoff:hoff + n])
        pa = (e[..., :-1] + e[..., 1:]) * 0.5 * sw[None]
        A = jnp.sum(pa, axis=-1)
        cwn = jnp.concatenate([cumw[:, 1:], cumw[:, -1:] + 1.0], axis=-1)
        if li == 0:
            cumw0_ref[...] = cumw
            cwn0_ref[...] = cwn
            a0_ref[...] = A[..., None]
        else:
            cumw1_ref[...] = cumw
            cwn1_ref[...] = cwn
            a1_ref[...] = A[..., None]
        woff += n - 1
        hoff += n


def _k1_call(uw, uh, dh):
    grid = (NG // BG1,)
    return pl.pallas_call(
        _k1_body,
        grid=grid,
        in_specs=[
            pl.BlockSpec((BG1, 94), lambda gb: (gb, 0)),
            pl.BlockSpec((BG1, 96), lambda gb: (gb, 0)),
            pl.BlockSpec((NR, BG1, 96), lambda gb: (0, gb, 0)),
        ],
        out_specs=[
            pl.BlockSpec((BG1, NB[0]), lambda gb: (gb, 0)),
            pl.BlockSpec((BG1, NB[1]), lambda gb: (gb, 0)),
            pl.BlockSpec((BG1, NB[0]), lambda gb: (gb, 0)),
            pl.BlockSpec((BG1, NB[1]), lambda gb: (gb, 0)),
            pl.BlockSpec((NR, BG1, 1), lambda gb: (0, gb, 0)),
            pl.BlockSpec((NR, BG1, 1), lambda gb: (0, gb, 0)),
        ],
        out_shape=[
            jax.ShapeDtypeStruct((NG, NB[0]), jnp.float32),
            jax.ShapeDtypeStruct((NG, NB[1]), jnp.float32),
            jax.ShapeDtypeStruct((NG, NB[0]), jnp.float32),
            jax.ShapeDtypeStruct((NG, NB[1]), jnp.float32),
            jax.ShapeDtypeStruct((NR, NG, 1), jnp.float32),
            jax.ShapeDtypeStruct((NR, NG, 1), jnp.float32),
        ],
    )(uw, uh, dh)


def _k2_body(a0_ref, a1_ref, cp0_ref, m0_ref, cp1_ref, s1_ref,
             ia0_ref, ia1_ref):
    A0 = a0_ref[...]
    GA0 = A0 * jnp.float32(1.0 / NG)
    area0 = jnp.sum(GA0, axis=-1, keepdims=True)
    M0 = GA0 / area0
    incl0 = _csum_last(M0)
    CP0 = jnp.concatenate(
        [jnp.zeros_like(M0[:, :1]), incl0[:, :-1]], axis=-1)
    A1 = a1_ref[...]
    G1 = M0
    GA1 = G1 * A1
    area1 = jnp.sum(GA1, axis=-1, keepdims=True)
    M1 = GA1 / area1
    incl1 = _csum_last(M1)
    CP1 = jnp.concatenate(
        [jnp.zeros_like(M1[:, :1]), incl1[:, :-1]], axis=-1)
    S1 = G1 / area1
    cp0_ref[...] = CP0
    m0_ref[...] = M0
    cp1_ref[...] = CP1
    s1_ref[...] = S1
    ia0_ref[...] = 1.0 / area0
    ia1_ref[...] = 1.0 / area1


def _k2_call(A0, A1):
    return pl.pallas_call(
        _k2_body,
        out_shape=[
            jax.ShapeDtypeStruct((NR, NG), jnp.float32),
            jax.ShapeDtypeStruct((NR, NG), jnp.float32),
            jax.ShapeDtypeStruct((NR, NG), jnp.float32),
            jax.ShapeDtypeStruct((NR, NG), jnp.float32),
            jax.ShapeDtypeStruct((NR, 1), jnp.float32),
            jax.ShapeDtypeStruct((NR, 1), jnp.float32),
        ],
    )(A0, A1)


def _k3_body(uw_ref, uh_ref, dh_ref,
             cp0_ref, s0_ref, cp1_ref, s1_ref, h0s_ref, h1s_ref,
             h0_ref, lc0_ref, h1_ref, lc1_ref):
    uw = uw_ref[...]
    uh = uh_ref[...]
    dh = dh_ref[...]
    woff = 0
    hoff = 0
    for li, n in enumerate(NB):
        uwl = uw[:, woff:woff + n - 1]
        sw = _softmax_last(uwl)
        e = jnp.exp(uh[:, hoff:hoff + n] + dh[:, hoff:hoff + n])
        pa = (e[:, :-1] + e[:, 1:]) * 0.5 * sw
        C = jnp.concatenate(
            [jnp.zeros_like(e[:, :1]), _csum_last(pa)], axis=-1)
        if li == 0:
            cp = cp0_ref[...]
            s = s0_ref[...]
            hs = h0s_ref[...]
        else:
            cp = cp1_ref[...]
            s = s1_ref[...]
            hs = h1s_ref[...]
        h = e * hs
        lc = cp + s * C
        if li == 0:
            h0_ref[...] = h
            lc0_ref[...] = lc
        else:
            h1_ref[...] = h
            lc1_ref[...] = lc
        woff += n - 1
        hoff += n


def _k3_call(uw, uh, dh, cp0x, s0x, cp1x, s1x, h0sx, h1sx):
    grid = (NR, NG // BG3)
    pgx = lambda r, gb: (gb, 0)
    prgx = lambda r, gb: (r, gb, 0)
    return pl.pallas_call(
        _k3_body,
        grid=grid,
        in_specs=[
            pl.BlockSpec((BG3, 94), pgx),
            pl.BlockSpec((BG3, 96), pgx),
            pl.BlockSpec((None, BG3, 96), prgx),
            pl.BlockSpec((None, BG3, 1), prgx),
            pl.BlockSpec((None, BG3, 1), prgx),
            pl.BlockSpec((None, BG3, 1), prgx),
            pl.BlockSpec((None, BG3, 1), prgx),
            pl.BlockSpec((None, BG3, 1), prgx),
            pl.BlockSpec((None, BG3, 1), prgx),
        ],
        out_specs=[
            pl.BlockSpec((None, BG3, NB[0]), prgx),
            pl.BlockSpec((None, BG3, NB[0]), prgx),
            pl.BlockSpec((None, BG3, NB[1]), prgx),
            pl.BlockSpec((None, BG3, NB[1]), prgx),
        ],
        out_shape=[
            jax.ShapeDtypeStruct((NR, NG, NB[0]), jnp.float32),
            jax.ShapeDtypeStruct((NR, NG, NB[0]), jnp.float32),
            jax.ShapeDtypeStruct((NR, NG, NB[1]), jnp.float32),
            jax.ShapeDtypeStruct((NR, NG, NB[1]), jnp.float32),
        ],
    )(uw, uh, dh, cp0x, s0x, cp1x, s1x, h0sx, h1sx)


def _sc_body(x_hbm, rxg_hbm, q0_hbm, q1_hbm, pg_hbm, out_hbm, dp_hbm,
             xv, rxgv, idxv, rows, pgv, outv, dpv, rows16, sem, sem2):
    c = lax.axis_index("c")
    s = lax.axis_index("s")
    wid = s * _NC + c
    tbase = wid * PER_TILE
    i16 = lax.iota(jnp.int32, 16)

    def fields_from(rref, lanes):
        return [
            plsc.load_gather(rref, [lanes, jnp.full((16,), ci, jnp.int32)])
            for ci in range(5)
        ]

    def level(lv, n, q_hbm):
        nm1 = jnp.float32(n - 1)
        jmaxi = n - 2

        def pg_of(gi, rxg):
            if lv == 0:
                gq = rxg % NG
                P = gq.astype(jnp.float32) * jnp.float32(1.0 / NG)
                G = jnp.full((16,), 1.0 / NG, jnp.float32)
            else:
                lanes = gi * 16 + i16
                P = plsc.load_gather(pgv, [lanes, jnp.zeros((16,), jnp.int32)])
                G = plsc.load_gather(pgv, [lanes, jnp.ones((16,), jnp.int32)])
            return P, G

        def guess(t):
            tj = jnp.clip(t * nm1, 0.0, jnp.float32(jmaxi))
            return tj.astype(jnp.int32)

        def passA(gi, carry):
            sl = pl.ds(gi * 16, 16)
            rxg = rxgv[sl]
            xx = xv[sl] if lv == 0 else outv[sl]
            P, G = pg_of(gi, rxg)
            t = (xx - P) / G
            j = guess(t)
            idxv[gi // 8, pl.ds((gi % 8) * 16, 16)] = rxg * n + j
            return carry

        lax.fori_loop(0, NGRP, passA, 0)

        cps = [
            pltpu.async_copy(q_hbm.at[idxv.at[si]],
                             rows.at[pl.ds(si * GSZ, GSZ)], sem)
            for si in range(NDMA)
        ]
        for cp_ in cps:
            cp_.wait()

        def passB(gi, carry):
            sl = pl.ds(gi * 16, 16)
            lanes = gi * 16 + i16
            rxg = rxgv[sl]
            xx = xv[sl] if lv == 0 else outv[sl]
            P, G = pg_of(gi, rxg)
            t = (xx - P) / G
            j = guess(t)
            cw, cw1, hj, hj1, lcv = fields_from(rows, lanes)

            def wr(j_, cw_, cw1_):
                return (((cw_ >= t) & (j_ > 0))
                        | ((cw1_ < t) & (j_ < jmaxi)))

            wrong = wr(j, cw, cw1)

            def cond(st):
                return jnp.any(st[6])

            def body(st):
                j_, cw_, cw1_, hj_, hj1_, lc_, w_ = st
                jn = jnp.where(w_, jnp.where(cw_ >= t, j_ - 1, j_ + 1), j_)
                pltpu.async_copy(q_hbm.at[rxg * n + jn], rows16, sem2).wait()
                nf = fields_from(rows16, i16)
                cwN = jnp.where(w_, nf[0], cw_)
                cw1N = jnp.where(w_, nf[1], cw1_)
                hjN = jnp.where(w_, nf[2], hj_)
                hj1N = jnp.where(w_, nf[3], hj1_)
                lcN = jnp.where(w_, nf[4], lc_)
                return (jn, cwN, cw1N, hjN, hj1N, lcN, wr(jn, cwN, cw1N))

            j, cw, cw1, hj, hj1, lcv, wrong = lax.while_loop(
                cond, body, (j, cw, cw1, hj, hj1, lcv, wrong))
            ll = P + G * cw
            bw = G * (cw1 - cw)
            alpha = jnp.clip(
                (xx - ll) / jnp.maximum(bw, jnp.float32(1e-12)), 0.0, 1.0)
            outn = (lcv + alpha * bw * hj
                    + jnp.float32(0.5) * alpha * alpha * bw * (hj1 - hj))
            dd = jnp.maximum(hj + alpha * (hj1 - hj), jnp.float32(1e-12))
            outv[sl] = outn
            if lv == 0:
                dpv[sl] = dd
            else:
                dpv[sl] = dpv[sl] * dd
            return carry

        lax.fori_loop(0, NGRP, passB, 0)

    def chunk(ci, carry):
        base = tbase + ci * CHUNK
        pltpu.sync_copy(x_hbm.at[pl.ds(base, CHUNK)], xv)
        pltpu.sync_copy(rxg_hbm.at[pl.ds(base, CHUNK)], rxgv)
        level(0, NB[0], q0_hbm)

        def pgidx(gi, carry):
            idxv[gi // 8, pl.ds((gi % 8) * 16, 16)] = rxgv[pl.ds(gi * 16, 16)]
            return carry

        lax.fori_loop(0, NGRP, pgidx, 0)
        cps = [
            pltpu.async_copy(pg_hbm.at[idxv.at[si]],
                             pgv.at[pl.ds(si * GSZ, GSZ)], sem)
            for si in range(NDMA)
        ]
        for cp_ in cps:
            cp_.wait()
        level(1, NB[1], q1_hbm)
        pltpu.sync_copy(outv, out_hbm.at[pl.ds(base, CHUNK)])
        pltpu.sync_copy(dpv, dp_hbm.at[pl.ds(base, CHUNK)])
        return carry

    lax.fori_loop(0, NCHUNK, chunk, 0)


def _sc_call(xp, rxgp, q0, q1, pg):
    mesh = plsc.VectorSubcoreMesh(core_axis_name="c", subcore_axis_name="s")
    f = pl.kernel(
        _sc_body,
        out_type=[
            jax.ShapeDtypeStruct((CUTS_PAD,), jnp.float32),
            jax.ShapeDtypeStruct((CUTS_PAD,), jnp.float32),
        ],
        mesh=mesh,
        compiler_params=pltpu.CompilerParams(
            needs_layout_passes=False, use_tc_tiling_on_sc=False),
        scratch_types=[
            pltpu.VMEM((CHUNK,), jnp.float32),
            pltpu.VMEM((CHUNK,), jnp.int32),
            pltpu.VMEM((NDMA, GSZ), jnp.int32),
            pltpu.VMEM((CHUNK, 8), jnp.float32),
            pltpu.VMEM((CHUNK, 8), jnp.float32),
            pltpu.VMEM((CHUNK,), jnp.float32),
            pltpu.VMEM((CHUNK,), jnp.float32),
            pltpu.VMEM((16, 8), jnp.float32),
            pltpu.SemaphoreType.DMA,
            pltpu.SemaphoreType.DMA,
        ],
    )
    return f(xp, rxgp, q0, q1, pg)


def _k4_body(dp_ref, lad_ref):
    lad_ref[...] = jnp.log(dp_ref[...])


def _k4_call(dp):
    d2 = dp.reshape(CUTS_PAD // 128, 128)
    out = pl.pallas_call(
        _k4_body,
        out_shape=jax.ShapeDtypeStruct((CUTS_PAD // 128, 128), jnp.float32),
    )(d2)
    return out.reshape(CUTS_PAD)


def kernel(cut_positions, cut_local_reflatentxgene_ix, cut_local_gene_ix,
           cut_local_reflatent_ix, mixture_delta_reflatentxgene,
           unnormalized_heights, unnormalized_widths):
    uw = unnormalized_widths
    uh = unnormalized_heights
    dh = mixture_delta_reflatentxgene

    cumw0, cumw1, cwn0, cwn1, A0x, A1x = _k1_call(uw, uh, dh)
    A0 = A0x[..., 0]
    A1 = A1x[..., 0]
    CP0, M0, CP1, S1, ia0, ia1 = _k2_call(A0, A1)

    cp0x = CP0[..., None]
    s0x = jnp.broadcast_to(ia0 * jnp.float32(1.0 / NG), (NR, NG))[..., None]
    cp1x = CP1[..., None]
    s1x = S1[..., None]
    h0sx = jnp.broadcast_to(ia0, (NR, NG))[..., None]
    h1sx = jnp.broadcast_to(ia1, (NR, NG))[..., None]

    h0, lc0, h1, lc1 = _k3_call(uw, uh, dh,
                                cp0x, s0x, cp1x, s1x, h0sx, h1sx)

    def _mk_q(cw, cwn, h, lc, n):
        hn = jnp.concatenate([h[..., 1:], h[..., -1:]], axis=-1)
        cwb = jnp.broadcast_to(cw[None], (NR, NG, n))
        cwnb = jnp.broadcast_to(cwn[None], (NR, NG, n))
        q = jnp.stack([cwb, cwnb, h, hn, lc], axis=-1)
        q = jnp.pad(q, ((0, 0), (0, 0), (0, 0), (0, 3)))
        return q.reshape(NR * NG * n, 8)

    q0 = _mk_q(cumw0, cwn0, h0, lc0, NB[0])
    q1 = _mk_q(cumw1, cwn1, h1, lc1, NB[1])
    pg = jnp.pad(
        jnp.stack([CP0.reshape(-1), M0.reshape(-1)], axis=-1),
        ((0, 0), (0, 6)))

    npad = CUTS_PAD - N_CUTS
    xp = jnp.pad(cut_positions, (0, npad))
    rxgp = jnp.pad(cut_local_reflatentxgene_ix, (0, npad))

    outp, dpp = _sc_call(xp, rxgp, q0, q1, pg)
    ladp = _k4_call(dpp)
    return outp[:N_CUTS], ladp[:N_CUTS]


# speedup vs baseline: 4.7959x; 3.7774x over previous
import functools

import jax
import jax.numpy as jnp
from jax import lax
from jax.experimental import pallas as pl
from jax.experimental.pallas import tpu as pltpu
from jax.experimental.pallas import tpu_sc as plsc

NB = (64, 32)
NG = 5000
NR = 16
N_CUTS = 500000

_NC = 2
_NS = 16
NW = _NC * _NS
CUTS_PAD = 512000
PER_TILE = CUTS_PAD // NW
CHUNK = 3200
NCHUNK = PER_TILE // CHUNK
GSZ = 128
GPR = GSZ // 16
NDMA = CHUNK // GSZ
NGRP = CHUNK // 16

BG1 = 200
BG3 = 200


def _csum_last(a):
    n = a.shape[-1]
    k = 1
    while k < n:
        sh = jnp.concatenate(
            [jnp.zeros_like(a[..., :k]), a[..., :-k]], axis=-1)
        a = a + sh
        k *= 2
    return a


def _softmax_last(u):
    m = jnp.max(u, axis=-1, keepdims=True)
    e = jnp.exp(u - m)
    return e / jnp.sum(e, axis=-1, keepdims=True)


def _k1_body(uw_ref, uh_ref, dh_ref, cumw0_ref, cumw1_ref, cwn0_ref,
             cwn1_ref, a0_ref, a1_ref):
    uw = uw_ref[...]
    uh = uh_ref[...]
    dh = dh_ref[...]
    woff = 0
    hoff = 0
    for li, n in enumerate(NB):
        uwl = uw[:, woff:woff + n - 1]
        sw = _softmax_last(uwl)
        incl = _csum_last(sw)
        cumw = jnp.concatenate(
            [jnp.zeros_like(sw[:, :1]), incl], axis=-1)
        e = jnp.exp(uh[None, :, hoff:hoff + n] + dh[:, :, hoff:hoff + n])
        pa = (e[..., :-1] + e[..., 1:]) * 0.5 * sw[None]
        A = jnp.sum(pa, axis=-1)
        cwn = jnp.concatenate([cumw[:, 1:], cumw[:, -1:] + 1.0], axis=-1)
        if li == 0:
            cumw0_ref[...] = cumw
            cwn0_ref[...] = cwn
            a0_ref[...] = A[..., None]
        else:
            cumw1_ref[...] = cumw
            cwn1_ref[...] = cwn
            a1_ref[...] = A[..., None]
        woff += n - 1
        hoff += n


def _k1_call(uw, uh, dh):
    grid = (NG // BG1,)
    return pl.pallas_call(
        _k1_body,
        grid=grid,
        in_specs=[
            pl.BlockSpec((BG1, 94), lambda gb: (gb, 0)),
            pl.BlockSpec((BG1, 96), lambda gb: (gb, 0)),
            pl.BlockSpec((NR, BG1, 96), lambda gb: (0, gb, 0)),
        ],
        out_specs=[
            pl.BlockSpec((BG1, NB[0]), lambda gb: (gb, 0)),
            pl.BlockSpec((BG1, NB[1]), lambda gb: (gb, 0)),
            pl.BlockSpec((BG1, NB[0]), lambda gb: (gb, 0)),
            pl.BlockSpec((BG1, NB[1]), lambda gb: (gb, 0)),
            pl.BlockSpec((NR, BG1, 1), lambda gb: (0, gb, 0)),
            pl.BlockSpec((NR, BG1, 1), lambda gb: (0, gb, 0)),
        ],
        out_shape=[
            jax.ShapeDtypeStruct((NG, NB[0]), jnp.float32),
            jax.ShapeDtypeStruct((NG, NB[1]), jnp.float32),
            jax.ShapeDtypeStruct((NG, NB[0]), jnp.float32),
            jax.ShapeDtypeStruct((NG, NB[1]), jnp.float32),
            jax.ShapeDtypeStruct((NR, NG, 1), jnp.float32),
            jax.ShapeDtypeStruct((NR, NG, 1), jnp.float32),
        ],
    )(uw, uh, dh)


def _k2_body(a0_ref, a1_ref, cp0_ref, m0_ref, cp1_ref, s1_ref,
             ia0_ref, ia1_ref):
    A0 = a0_ref[...]
    GA0 = A0 * jnp.float32(1.0 / NG)
    area0 = jnp.sum(GA0, axis=-1, keepdims=True)
    M0 = GA0 / area0
    incl0 = _csum_last(M0)
    CP0 = jnp.concatenate(
        [jnp.zeros_like(M0[:, :1]), incl0[:, :-1]], axis=-1)
    A1 = a1_ref[...]
    G1 = M0
    GA1 = G1 * A1
    area1 = jnp.sum(GA1, axis=-1, keepdims=True)
    M1 = GA1 / area1
    incl1 = _csum_last(M1)
    CP1 = jnp.concatenate(
        [jnp.zeros_like(M1[:, :1]), incl1[:, :-1]], axis=-1)
    S1 = G1 / area1
    cp0_ref[...] = CP0
    m0_ref[...] = M0
    cp1_ref[...] = CP1
    s1_ref[...] = S1
    ia0_ref[...] = 1.0 / area0
    ia1_ref[...] = 1.0 / area1


def _k2_call(A0, A1):
    return pl.pallas_call(
        _k2_body,
        out_shape=[
            jax.ShapeDtypeStruct((NR, NG), jnp.float32),
            jax.ShapeDtypeStruct((NR, NG), jnp.float32),
            jax.ShapeDtypeStruct((NR, NG), jnp.float32),
            jax.ShapeDtypeStruct((NR, NG), jnp.float32),
            jax.ShapeDtypeStruct((NR, 1), jnp.float32),
            jax.ShapeDtypeStruct((NR, 1), jnp.float32),
        ],
    )(A0, A1)


def _k3_body(uw_ref, uh_ref, dh_ref, cumw0_ref, cumw1_ref,
             cwn0_ref, cwn1_ref,
             cp0_ref, s0_ref, cp1_ref, s1_ref, h0s_ref, h1s_ref,
             q0_ref, q1_ref):
    uw = uw_ref[...]
    uh = uh_ref[...]
    dh = dh_ref[...]
    woff = 0
    hoff = 0
    for li, n in enumerate(NB):
        uwl = uw[:, woff:woff + n - 1]
        sw = _softmax_last(uwl)
        e = jnp.exp(uh[:, hoff:hoff + n] + dh[:, hoff:hoff + n])
        pa = (e[:, :-1] + e[:, 1:]) * 0.5 * sw
        C = jnp.concatenate(
            [jnp.zeros_like(e[:, :1]), _csum_last(pa)], axis=-1)
        if li == 0:
            cw = cumw0_ref[...]
            cwn = cwn0_ref[...]
            cp = cp0_ref[...]
            s = s0_ref[...]
            hs = h0s_ref[...]
        else:
            cw = cumw1_ref[...]
            cwn = cwn1_ref[...]
            cp = cp1_ref[...]
            s = s1_ref[...]
            hs = h1s_ref[...]
        h = e * hs
        lc = cp + s * C
        hn = jnp.concatenate([h[:, 1:], h[:, -1:]], axis=-1)
        z16 = jnp.zeros((BG3, 48), jnp.float32)
        ii = lax.broadcasted_iota(jnp.int32, (128, 128), 0)
        mm = lax.broadcasted_iota(jnp.int32, (128, 128), 1)
        PM = (ii == (mm % 8) * 16 + mm // 8).astype(jnp.float32)
        vs = []
        for t in range(n // 16):
            sl = slice(t * 16, t * 16 + 16)
            fct = jnp.concatenate(
                [cw[:, sl], cwn[:, sl], h[:, sl], hn[:, sl], lc[:, sl], z16],
                axis=-1)
            vs.append(lax.dot_general(
                fct, PM, (((1,), (0,)), ((), ())),
                precision=lax.Precision.HIGHEST,
                preferred_element_type=jnp.float32))
        q = jnp.concatenate(vs, axis=0)
        if li == 0:
            q0_ref[...] = q
        else:
            q1_ref[...] = q
        woff += n - 1
        hoff += n


def _k3_call(uw, uh, dh, cumw0, cumw1, cwn0, cwn1,
             cp0x, s0x, cp1x, s1x, h0sx, h1sx):
    grid = (NR, NG // BG3)
    pgx = lambda r, gb: (gb, 0)
    prgx = lambda r, gb: (r, gb, 0)
    pfx = lambda r, gb: (r * (NG // BG3) + gb, 0)
    return pl.pallas_call(
        _k3_body,
        grid=grid,
        in_specs=[
            pl.BlockSpec((BG3, 94), pgx),
            pl.BlockSpec((BG3, 96), pgx),
            pl.BlockSpec((None, BG3, 96), prgx),
            pl.BlockSpec((BG3, NB[0]), pgx),
            pl.BlockSpec((BG3, NB[1]), pgx),
            pl.BlockSpec((BG3, NB[0]), pgx),
            pl.BlockSpec((BG3, NB[1]), pgx),
            pl.BlockSpec((None, BG3, 1), prgx),
            pl.BlockSpec((None, BG3, 1), prgx),
            pl.BlockSpec((None, BG3, 1), prgx),
            pl.BlockSpec((None, BG3, 1), prgx),
            pl.BlockSpec((None, BG3, 1), prgx),
            pl.BlockSpec((None, BG3, 1), prgx),
        ],
        out_specs=[
            pl.BlockSpec((BG3 * NB[0] * 8 // 128, 128), pfx),
            pl.BlockSpec((BG3 * NB[1] * 8 // 128, 128), pfx),
        ],
        out_shape=[
            jax.ShapeDtypeStruct((NR * NG * NB[0] * 8 // 128, 128),
                                 jnp.float32),
            jax.ShapeDtypeStruct((NR * NG * NB[1] * 8 // 128, 128),
                                 jnp.float32),
        ],
    )(uw, uh, dh, cumw0, cumw1, cwn0, cwn1,
      cp0x, s0x, cp1x, s1x, h0sx, h1sx)


def _sc_body(x_hbm, rxg_hbm, qb0_hbm, qb1_hbm, p0_hbm,
             q0_hbm, q1_hbm, pg_hbm, out_hbm, dp_hbm,
             xv, rxgv, qb0v, qb1v, p0v, idxv, rows, pgv, outv, dpv,
             rows16, sem, sem2):
    c = lax.axis_index("c")
    s = lax.axis_index("s")
    wid = s * _NC + c
    tbase = wid * PER_TILE
    i16 = lax.iota(jnp.int32, 16)

    def fields_from(rref, rowix):
        return [
            plsc.load_gather(rref, [rowix, jnp.full((16,), ci, jnp.int32)])
            for ci in range(5)
        ]

    def level(lv, n, q_hbm):
        nm1 = jnp.float32(n - 1)
        jmaxi = n - 2
        tpg = n // 16

        qbv = qb0v if lv == 0 else qb1v

        def rowof(qb, j):
            return (qb + jnp.right_shift(j, 4) * (BG3 * 16)
                    + jnp.bitwise_and(j, 15))

        def pg_of(gi, rxg):
            if lv == 0:
                P = p0v[pl.ds(gi * 16, 16)]
                G = jnp.full((16,), 1.0 / NG, jnp.float32)
            else:
                lanes = gi * 16 + i16
                P = plsc.load_gather(pgv, [lanes, jnp.zeros((16,), jnp.int32)])
                G = plsc.load_gather(pgv, [lanes, jnp.ones((16,), jnp.int32)])
            return P, G

        def guess(t):
            tj = jnp.clip(t * nm1, 0.0, jnp.float32(jmaxi))
            return tj.astype(jnp.int32)

        def passA(gi, carry):
            sl = pl.ds(gi * 16, 16)
            rxg = rxgv[sl]
            xx = xv[sl] if lv == 0 else outv[sl]
            P, G = pg_of(gi, rxg)
            t = (xx - P) / G
            j = guess(t)
            qb = qbv[sl]
            idxv[gi // GPR, pl.ds((gi % GPR) * 16, 16)] = rowof(qb, j)
            return carry

        lax.fori_loop(0, NGRP, passA, 0)

        cps = [
            pltpu.async_copy(q_hbm.at[idxv.at[si]],
                             rows.at[pl.ds(si * GSZ, GSZ)], sem)
            for si in range(NDMA)
        ]
        for cp_ in cps:
            cp_.wait()

        def passB(gi, carry):
            sl = pl.ds(gi * 16, 16)
            lanes = gi * 16 + i16
            rxg = rxgv[sl]
            xx = xv[sl] if lv == 0 else outv[sl]
            P, G = pg_of(gi, rxg)
            t = (xx - P) / G
            j = guess(t)
            qb = qbv[sl]
            cw, cw1, hj, hj1, lcv = fields_from(rows, lanes)

            def wr(j_, cw_, cw1_):
                return (((cw_ >= t) & (j_ > 0))
                        | ((cw1_ < t) & (j_ < jmaxi)))

            wrong = wr(j, cw, cw1)

            def cond(st):
                return jnp.any(st[6])

            def body(st):
                j_, cw_, cw1_, hj_, hj1_, lc_, w_ = st
                jn = jnp.where(w_, jnp.where(cw_ >= t, j_ - 1, j_ + 1), j_)
                pltpu.async_copy(q_hbm.at[rowof(qb, jn)], rows16, sem2).wait()
                nf = fields_from(rows16, i16)
                cwN = jnp.where(w_, nf[0], cw_)
                cw1N = jnp.where(w_, nf[1], cw1_)
                hjN = jnp.where(w_, nf[2], hj_)
                hj1N = jnp.where(w_, nf[3], hj1_)
                lcN = jnp.where(w_, nf[4], lc_)
                return (jn, cwN, cw1N, hjN, hj1N, lcN, wr(jn, cwN, cw1N))

            j, cw, cw1, hj, hj1, lcv, wrong = lax.while_loop(
                cond, body, (j, cw, cw1, hj, hj1, lcv, wrong))
            ll = P + G * cw
            bw = G * (cw1 - cw)
            alpha = jnp.clip(
                (xx - ll) / jnp.maximum(bw, jnp.float32(1e-12)), 0.0, 1.0)
            outn = (lcv + alpha * bw * hj
                    + jnp.float32(0.5) * alpha * alpha * bw * (hj1 - hj))
            dd = jnp.maximum(hj + alpha * (hj1 - hj), jnp.float32(1e-12))
            outv[sl] = outn
            if lv == 0:
                dpv[sl] = dd
            else:
                dpv[sl] = dpv[sl] * dd
            return carry

        lax.fori_loop(0, NGRP, passB, 0)

    def chunk(ci, carry):
        base = tbase + ci * CHUNK
        pltpu.sync_copy(x_hbm.at[pl.ds(base, CHUNK)], xv)
        pltpu.sync_copy(rxg_hbm.at[pl.ds(base, CHUNK)], rxgv)
        pltpu.sync_copy(qb0_hbm.at[pl.ds(base, CHUNK)], qb0v)
        pltpu.sync_copy(qb1_hbm.at[pl.ds(base, CHUNK)], qb1v)
        pltpu.sync_copy(p0_hbm.at[pl.ds(base, CHUNK)], p0v)
        level(0, NB[0], q0_hbm)

        def pgidx(gi, carry):
            rxg = rxgv[pl.ds(gi * 16, 16)]
            idxv[gi // GPR, pl.ds((gi % GPR) * 16, 16)] = rxg
            return carry

        lax.fori_loop(0, NGRP, pgidx, 0)
        cps = [
            pltpu.async_copy(pg_hbm.at[idxv.at[si]],
                             pgv.at[pl.ds(si * GSZ, GSZ)], sem)
            for si in range(NDMA)
        ]
        for cp_ in cps:
            cp_.wait()
        level(1, NB[1], q1_hbm)
        pltpu.sync_copy(outv, out_hbm.at[pl.ds(base, CHUNK)])
        pltpu.sync_copy(dpv, dp_hbm.at[pl.ds(base, CHUNK)])
        return carry

    lax.fori_loop(0, NCHUNK, chunk, 0)


def _sc_call(xp, rxgp, qb0p, qb1p, p0p, q0, q1, pg):
    mesh = plsc.VectorSubcoreMesh(core_axis_name="c", subcore_axis_name="s")
    f = pl.kernel(
        _sc_body,
        out_type=[
            jax.ShapeDtypeStruct((CUTS_PAD,), jnp.float32),
            jax.ShapeDtypeStruct((CUTS_PAD,), jnp.float32),
        ],
        mesh=mesh,
        compiler_params=pltpu.CompilerParams(
            needs_layout_passes=False, use_tc_tiling_on_sc=False),
        scratch_types=[
            pltpu.VMEM((CHUNK,), jnp.float32),
            pltpu.VMEM((CHUNK,), jnp.int32),
            pltpu.VMEM((CHUNK,), jnp.int32),
            pltpu.VMEM((CHUNK,), jnp.int32),
            pltpu.VMEM((CHUNK,), jnp.float32),
            pltpu.VMEM((NDMA, GSZ), jnp.int32),
            pltpu.VMEM((CHUNK, 8), jnp.float32),
            pltpu.VMEM((CHUNK, 8), jnp.float32),
            pltpu.VMEM((CHUNK,), jnp.float32),
            pltpu.VMEM((CHUNK,), jnp.float32),
            pltpu.VMEM((16, 8), jnp.float32),
            pltpu.SemaphoreType.DMA,
            pltpu.SemaphoreType.DMA,
        ],
    )
    return f(xp, rxgp, qb0p, qb1p, p0p, q0, q1, pg)


def _k4_body(dp_ref, lad_ref):
    lad_ref[...] = jnp.log(dp_ref[...])


def _k4_call(dp):
    d2 = dp.reshape(CUTS_PAD // 128, 128)
    out = pl.pallas_call(
        _k4_body,
        out_shape=jax.ShapeDtypeStruct((CUTS_PAD // 128, 128), jnp.float32),
    )(d2)
    return out.reshape(CUTS_PAD)


def kernel(cut_positions, cut_local_reflatentxgene_ix, cut_local_gene_ix,
           cut_local_reflatent_ix, mixture_delta_reflatentxgene,
           unnormalized_heights, unnormalized_widths):
    uw = unnormalized_widths
    uh = unnormalized_heights
    dh = mixture_delta_reflatentxgene

    cumw0, cumw1, cwn0, cwn1, A0x, A1x = _k1_call(uw, uh, dh)
    A0 = A0x[..., 0]
    A1 = A1x[..., 0]
    CP0, M0, CP1, S1, ia0, ia1 = _k2_call(A0, A1)

    cp0x = CP0[..., None]
    s0x = jnp.broadcast_to(ia0 * jnp.float32(1.0 / NG), (NR, NG))[..., None]
    cp1x = CP1[..., None]
    s1x = S1[..., None]
    h0sx = jnp.broadcast_to(ia0, (NR, NG))[..., None]
    h1sx = jnp.broadcast_to(ia1, (NR, NG))[..., None]

    q0w, q1w = _k3_call(uw, uh, dh, cumw0, cumw1, cwn0, cwn1,
                        cp0x, s0x, cp1x, s1x, h0sx, h1sx)
    q0 = q0w.reshape(NR * NG * NB[0], 8)
    q1 = q1w.reshape(NR * NG * NB[1], 8)
    pg = jnp.pad(
        jnp.stack([CP0.reshape(-1), M0.reshape(-1)], axis=-1),
        ((0, 0), (0, 6)))

    gix = cut_local_gene_ix
    rix = cut_local_reflatent_ix
    gdiv = gix // BG3
    gmod = gix % BG3
    blk = rix * (NG // BG3) + gdiv
    qb0 = ((blk * (NB[0] // 16) + 0) * BG3 + gmod) * 16
    qb1 = ((blk * (NB[1] // 16) + 0) * BG3 + gmod) * 16
    p0 = gix.astype(jnp.float32) * jnp.float32(1.0 / NG)

    npad = CUTS_PAD - N_CUTS
    xp = jnp.pad(cut_positions, (0, npad))
    rxgp = jnp.pad(cut_local_reflatentxgene_ix, (0, npad))
    qb0p = jnp.pad(qb0, (0, npad))
    qb1p = jnp.pad(qb1, (0, npad))
    p0p = jnp.pad(p0, (0, npad))

    outp, dpp = _sc_call(xp, rxgp, qb0p, qb1p, p0p, q0, q1, pg)
    ladp = _k4_call(dpp)
    return outp[:N_CUTS], ladp[:N_CUTS]
